# baseline (speedup 1.0000x reference)
import functools
import math

import jax
import jax.numpy as jnp
from jax import lax
from jax.experimental import pallas as pl
from jax.experimental.pallas import tpu as pltpu
from jax.experimental.pallas import tpu_sc as plsc

D_MODEL = 64
SCALE = math.sqrt(D_MODEL)
LANES = 16
CCHUNK = 128
NBUF = 2


@functools.lru_cache(maxsize=None)
def _build(b_total):
    info = plsc.get_sparse_core_info()
    nc, ns = info.num_cores, info.num_subcores
    nw = nc * ns
    b_per_w = b_total // nw
    nchunk = b_per_w // CCHUNK
    assert b_per_w * nw == b_total and nchunk * CCHUNK == b_per_w

    mesh = plsc.VectorSubcoreMesh(core_axis_name="c", subcore_axis_name="s")

    @functools.partial(
        pl.kernel,
        mesh=mesh,
        compiler_params=pltpu.CompilerParams(use_tc_tiling_on_sc=False),
        out_type=jax.ShapeDtypeStruct((b_total, D_MODEL), jnp.float32),
        scratch_types=[
            pltpu.VMEM((nchunk, CCHUNK), jnp.int32),
            pltpu.VMEM((NBUF, CCHUNK, D_MODEL), jnp.float32),
            pltpu.VMEM((NBUF, CCHUNK, D_MODEL), jnp.float32),
            pltpu.SemaphoreType.DMA,
            pltpu.SemaphoreType.DMA,
            pltpu.SemaphoreType.DMA,
            pltpu.SemaphoreType.DMA,
        ],
    )
    def emb_kernel(x_hbm, table_hbm, out_hbm, idx_v, gbuf, obuf,
                   gs0, gs1, os0, os1):
        gsems = (gs0, gs1)
        osems = (os0, os1)
        wid = lax.axis_index("s") * nc + lax.axis_index("c")
        row_base = wid * b_per_w

        pltpu.sync_copy(x_hbm.at[wid], idx_v)

        def gather(c, b):
            pltpu.async_copy(table_hbm.at[idx_v.at[c]], gbuf.at[b], gsems[b])

        def gwait(c, b):
            pltpu.make_async_copy(
                table_hbm.at[idx_v.at[c]], gbuf.at[b], gsems[b]).wait()

        def out_start(c, b):
            pltpu.async_copy(
                obuf.at[b],
                out_hbm.at[pl.ds(row_base + c * CCHUNK, CCHUNK)],
                osems[b])

        def owait(c, b):
            pltpu.make_async_copy(
                obuf.at[b],
                out_hbm.at[pl.ds(row_base + c * CCHUNK, CCHUNK)],
                osems[b]).wait()

        def scale(b):
            def body(r, carry):
                for d in range(D_MODEL // LANES):
                    v = gbuf[b, r, pl.ds(d * LANES, LANES)]
                    obuf[b, r, pl.ds(d * LANES, LANES)] = v * SCALE
                return carry
            lax.fori_loop(0, CCHUNK, body, 0)

        for b in range(NBUF):
            gather(b, b)
        for b in range(NBUF):
            gwait(b, b)
            scale(b)
            out_start(b, b)
            gather(b + NBUF, b)

        def chunk_pair(i, carry):
            for b in range(NBUF):
                c = i * NBUF + b
                gwait(c, b)
                owait(c - NBUF, b)
                scale(b)
                out_start(c, b)

                @pl.when(c + NBUF < nchunk)
                def _():
                    gather(c + NBUF, b)
            return carry

        lax.fori_loop(1, nchunk // NBUF, chunk_pair, 0)

        for b in range(NBUF):
            owait(nchunk - NBUF + b, b)

    return emb_kernel, nw


def kernel(x, table):
    b_total = x.shape[0] * x.shape[1]
    emb, nw = _build(b_total)
    x_flat = x.reshape(nw, -1, CCHUNK)
    out = emb(x_flat, table)
    return out.reshape(x.shape[0], x.shape[1], D_MODEL)
